# baseline (speedup 1.0000x reference)
import functools
import jax
import jax.numpy as jnp
from jax import lax
from jax.experimental import pallas as pl
from jax.experimental.pallas import tpu as pltpu
from jax.experimental.pallas import tpu_sc as plsc

_N_UIDS = 8192
_TOPK = 20
_UID_BLK = 1024
_ROW_BLK = 128


def _gate_body(q_ref, w_ref, b_ref, o_ref):
    mq = jnp.mean(q_ref[...], axis=0)
    prod = w_ref[...] * mq[None, None, :]
    o_ref[...] = jnp.sum(prod, axis=2) + b_ref[...]


def _gate_stage(query, gate_W, gate_b):
    nblk = _N_UIDS // _UID_BLK
    w3 = gate_W.reshape(_N_UIDS // 128, 128, gate_W.shape[1])
    b2 = gate_b.reshape(_N_UIDS // 128, 128)
    out = pl.pallas_call(
        _gate_body,
        grid=(nblk,),
        in_specs=[
            pl.BlockSpec(query.shape, lambda i: (0, 0)),
            pl.BlockSpec((_UID_BLK // 128, 128, gate_W.shape[1]),
                         lambda i: (i, 0, 0)),
            pl.BlockSpec((_UID_BLK // 128, 128), lambda i: (i, 0)),
        ],
        out_specs=pl.BlockSpec((_UID_BLK // 128, 128), lambda i: (i, 0)),
        out_shape=jax.ShapeDtypeStruct((_N_UIDS // 128, 128), jnp.float32),
    )(query, w3, b2)
    return out


def _topk_body(mw_ref, tw_ref, ow_ref, rs_ref):
    vals = mw_ref[...]
    ridx = jax.lax.broadcasted_iota(jnp.int32, vals.shape, 0)
    cidx = jax.lax.broadcasted_iota(jnp.int32, vals.shape, 1)
    flat = ridx * 128 + cidx
    big = jnp.int32(2 ** 30)
    tvals, tidxs = [], []
    for _ in range(_TOPK):
        m = jnp.max(vals)
        i = jnp.min(jnp.where(vals == m, flat, big))
        tvals.append(m)
        tidxs.append(i)
        vals = jnp.where(flat == i, -jnp.inf, vals)

    lane = jax.lax.broadcasted_iota(jnp.int32, (1, 128), 1)
    tw = jnp.zeros((1, 128), jnp.float32)
    for r in range(_TOPK):
        tw = jnp.where(lane == r, tvals[r], tw)
    tw_ref[...] = tw

    mn = tvals[-1]
    total = tvals[0] - mn
    for r in range(1, _TOPK):
        total = total + (tvals[r] - mn)
    ow = jnp.zeros(vals.shape, jnp.float32)
    member = jnp.zeros(vals.shape, jnp.bool_)
    for r in range(_TOPK):
        hit = flat == tidxs[r]
        ow = jnp.where(hit, (tvals[r] - mn) / total, ow)
        member = jnp.logical_or(member, hit)
    ow_ref[...] = ow
    rs_ref[...] = jnp.where(member, jnp.float32(32.0), jnp.float32(0.0))


def _topk_stage(mw):
    tw, ow, rs = pl.pallas_call(
        _topk_body,
        out_shape=[
            jax.ShapeDtypeStruct((1, 128), jnp.float32),
            jax.ShapeDtypeStruct(mw.shape, jnp.float32),
            jax.ShapeDtypeStruct(mw.shape, jnp.float32),
        ],
    )(mw)
    return tw, ow, rs


_SC_CHUNK = 512
_SC_NV = _SC_CHUNK // 16

_NEG = float("-inf")
_BIGI = 2 ** 30


def _sc_topk_rounds(vals_ref, idxs_ref, nv):
    lane = lax.iota(jnp.int32, 16)

    def round_body(r, carry):
        rv0, rv1, ri0, ri1 = carry
        mv = vals_ref[pl.ds(0, 16)]
        mi = idxs_ref[pl.ds(0, 16)]
        mp = lane
        for j in range(1, nv):
            bv = vals_ref[pl.ds(j * 16, 16)]
            bi = idxs_ref[pl.ds(j * 16, 16)]
            bp = lane + j * 16
            t = (bv > mv) | ((bv == mv) & (bi < mi))
            mv = jnp.where(t, bv, mv)
            mi = jnp.where(t, bi, mi)
            mp = jnp.where(t, bp, mp)
        m = jnp.max(mv)
        sel = mv == m
        i = jnp.min(jnp.where(sel, mi, _BIGI))
        p = jnp.min(jnp.where(sel & (mi == i), mp, _BIGI))
        in0 = lane == r
        in1 = lane == (r - 16)
        rv0 = jnp.where(in0, m, rv0)
        ri0 = jnp.where(in0, i, ri0)
        rv1 = jnp.where(in1, m, rv1)
        ri1 = jnp.where(in1, i, ri1)
        plsc.store_scatter(vals_ref, [jnp.full((16,), p, jnp.int32)],
                           jnp.full((16,), _NEG, jnp.float32),
                           mask=lane == 0)
        return rv0, rv1, ri0, ri1

    init = (jnp.full((16,), _NEG, jnp.float32),
            jnp.full((16,), _NEG, jnp.float32),
            jnp.full((16,), _BIGI, jnp.int32),
            jnp.full((16,), _BIGI, jnp.int32))
    return lax.fori_loop(0, _TOPK, round_body, init)


def _sc_body(mw_hbm, tw_hbm, ow_hbm, rs_hbm,
             vals_v, idxs_v, cand_v, cand_i, shv, shi, fshv, fshi,
             sb_v, sb_i, ow_t, rs_t):
    cid = lax.axis_index("c")
    sid = lax.axis_index("s")
    active = cid == 0
    lane = lax.iota(jnp.int32, 16)
    gbase = sid * _SC_CHUNK

    @pl.when(active)
    def _level0():
        pltpu.sync_copy(mw_hbm.at[pl.ds(gbase, _SC_CHUNK)], vals_v)
        for j in range(_SC_NV):
            idxs_v[pl.ds(j * 16, 16)] = lane + (gbase + j * 16)
        rv0, rv1, ri0, ri1 = _sc_topk_rounds(vals_v, idxs_v, _SC_NV)
        cand_v[pl.ds(0, 16)] = rv0
        cand_v[pl.ds(16, 16)] = rv1
        cand_i[pl.ds(0, 16)] = ri0
        cand_i[pl.ds(16, 16)] = ri1
        pltpu.sync_copy(cand_v, shv.at[pl.ds(sid * 32, 32)])
        pltpu.sync_copy(cand_i, shi.at[pl.ds(sid * 32, 32)])

    plsc.subcore_barrier()

    @pl.when(active & (sid == 0))
    def _level1():
        pltpu.sync_copy(shv, vals_v)
        pltpu.sync_copy(shi, idxs_v)
        fv0, fv1, fi0, fi1 = _sc_topk_rounds(vals_v, idxs_v, _SC_NV)
        valid1 = lane < (_TOPK - 16)
        mn = jnp.minimum(jnp.min(fv0),
                         jnp.min(jnp.where(valid1, fv1,
                                           jnp.float32(float("inf")))))
        total = (jnp.sum(fv0 - mn)
                 + jnp.sum(jnp.where(valid1, fv1 - mn, 0.0)))
        s0 = (fv0 - mn) / total
        s1 = jnp.where(valid1, (fv1 - mn) / total, 0.0)
        cand_v[pl.ds(0, 16)] = fv0
        cand_v[pl.ds(16, 16)] = fv1
        pltpu.sync_copy(cand_v, tw_hbm)
        cand_v[pl.ds(0, 16)] = s0
        cand_v[pl.ds(16, 16)] = s1
        cand_i[pl.ds(0, 16)] = fi0
        cand_i[pl.ds(16, 16)] = jnp.where(valid1, fi1, _BIGI)
        pltpu.sync_copy(cand_v, fshv)
        pltpu.sync_copy(cand_i, fshi)

    plsc.subcore_barrier()

    @pl.when(active)
    def _scatter():
        pltpu.sync_copy(fshv, sb_v)
        pltpu.sync_copy(fshi, sb_i)
        s0 = sb_v[pl.ds(0, 16)]
        s1 = sb_v[pl.ds(16, 16)]
        i0 = sb_i[pl.ds(0, 16)]
        i1 = sb_i[pl.ds(16, 16)]
        zero = jnp.zeros((16,), jnp.float32)
        for j in range(_SC_NV):
            ow_t[pl.ds(j * 16, 16)] = zero
            rs_t[pl.ds(j * 16, 16)] = zero
        m0 = (i0 >= gbase) & (i0 < gbase + _SC_CHUNK)
        m1 = (i1 >= gbase) & (i1 < gbase + _SC_CHUNK)
        p0 = jnp.clip(i0 - gbase, 0, _SC_CHUNK - 1)
        p1 = jnp.clip(i1 - gbase, 0, _SC_CHUNK - 1)
        bsz = jnp.full((16,), 32.0, jnp.float32)
        plsc.store_scatter(ow_t, [p0], s0, mask=m0)
        plsc.store_scatter(ow_t, [p1], s1, mask=m1)
        plsc.store_scatter(rs_t, [p0], bsz, mask=m0)
        plsc.store_scatter(rs_t, [p1], bsz, mask=m1)
        pltpu.sync_copy(ow_t, ow_hbm.at[pl.ds(gbase, _SC_CHUNK)])
        pltpu.sync_copy(rs_t, rs_hbm.at[pl.ds(gbase, _SC_CHUNK)])


def _sc_topk_stage(mw_flat):
    f32 = jnp.float32
    i32 = jnp.int32
    run = pl.kernel(
        _sc_body,
        mesh=plsc.VectorSubcoreMesh(core_axis_name="c", subcore_axis_name="s"),
        compiler_params=pltpu.CompilerParams(needs_layout_passes=False),
        out_type=[
            jax.ShapeDtypeStruct((32,), f32),
            jax.ShapeDtypeStruct((_N_UIDS,), f32),
            jax.ShapeDtypeStruct((_N_UIDS,), f32),
        ],
        scratch_types=[
            pltpu.VMEM((_SC_CHUNK,), f32),
            pltpu.VMEM((_SC_CHUNK,), i32),
            pltpu.VMEM((32,), f32),
            pltpu.VMEM((32,), i32),
            pltpu.VMEM_SHARED((512,), f32),
            pltpu.VMEM_SHARED((512,), i32),
            pltpu.VMEM_SHARED((32,), f32),
            pltpu.VMEM_SHARED((32,), i32),
            pltpu.VMEM((32,), f32),
            pltpu.VMEM((32,), i32),
            pltpu.VMEM((_SC_CHUNK,), f32),
            pltpu.VMEM((_SC_CHUNK,), f32),
        ],
    )
    return run(mw_flat)


def _join_body(w_ref, r_ref, o_ref):
    acc = r_ref[0] * w_ref[0]
    for i in range(1, _TOPK):
        acc = acc + r_ref[i] * w_ref[i]
    o_ref[...] = acc


def _join_stage(tw20, responses):
    k, b, s, d = responses.shape
    rows = b * s
    r3 = responses.reshape(k, rows, d)
    out = pl.pallas_call(
        _join_body,
        grid=(rows // _ROW_BLK,),
        in_specs=[
            pl.BlockSpec(memory_space=pltpu.SMEM),
            pl.BlockSpec((k, _ROW_BLK, d), lambda i: (0, i, 0)),
        ],
        out_specs=pl.BlockSpec((_ROW_BLK, d), lambda i: (i, 0)),
        out_shape=jax.ShapeDtypeStruct((rows, d), jnp.float32),
    )(tw20, r3)
    return out.reshape(b, s, d)


def kernel(query, responses, gate_W, gate_b):
    mw = _gate_stage(query, gate_W, gate_b)
    tw, ow, rs = _sc_topk_stage(mw.reshape(_N_UIDS))
    weighted = _join_stage(tw[:_TOPK], responses)
    return weighted, ow, rs
